# Initial kernel scaffold; baseline (speedup 1.0000x reference)
import jax
import jax.numpy as jnp
from jax.experimental import pallas as pl


def kernel(node_scalars, edge_feats, d, edge_index, W1, b1, W2, b2, gamma, beta):
    raise NotImplementedError("write your pallas kernel here")


# speedup vs baseline: 1.9677x; 1.9677x over previous
import functools

import jax
import jax.numpy as jnp
from jax.experimental import pallas as pl
from jax.experimental.pallas import tpu as pltpu
from jax.experimental.pallas import tpu_sc as plsc

_WINDOW = 128
_WORKERS = 32
_BE = 2000


def _sc_gather(table, idx2d, npad, dim):
    mesh = plsc.VectorSubcoreMesh(core_axis_name="core", subcore_axis_name="subcore")

    @functools.partial(
        pl.kernel,
        out_type=jax.ShapeDtypeStruct((npad, dim), table.dtype),
        mesh=mesh,
        compiler_params=pltpu.CompilerParams(use_tc_tiling_on_sc=False),
    )
    def gather_kernel(x_hbm, i_hbm, o_hbm):
        def body(i_vmem, o_vmem):
            pltpu.sync_copy(x_hbm.at[i_vmem.at[0]], o_vmem)

        pltpu.emit_pipeline(
            body,
            grid=(npad // _WINDOW,),
            in_specs=[pl.BlockSpec((1, _WINDOW), index_map=lambda i: (0, i))],
            out_specs=[pl.BlockSpec((_WINDOW, dim), index_map=lambda i: (i, 0))],
            core_axis_name=("core", "subcore"),
            dimension_semantics=(pltpu.PARALLEL,),
        )(i_hbm, o_hbm)

    return gather_kernel(table, idx2d)


def _mlp_body(gs_ref, gd_ref, ef_ref, d_ref, w1_ref, b1_ref, w2_ref, b2_ref,
              gamma_ref, beta_ref, out_ref):
    ef = ef_ref[...]
    x1 = (
        jnp.dot(gs_ref[...], w1_ref[0:64, :], preferred_element_type=jnp.float32)
        + jnp.dot(gd_ref[...], w1_ref[64:128, :], preferred_element_type=jnp.float32)
        + jnp.dot(ef, w1_ref[128:192, :], preferred_element_type=jnp.float32)
        + jnp.dot(d_ref[...], w1_ref[192:208, :], preferred_element_type=jnp.float32)
        + b1_ref[...]
    )
    h = x1 * jax.nn.sigmoid(x1)
    x2 = jnp.dot(h, w2_ref[...], preferred_element_type=jnp.float32) + b2_ref[...]
    x = ef + x2 * jax.nn.sigmoid(x2)
    mu = jnp.mean(x, axis=-1, keepdims=True)
    xc = x - mu
    var = jnp.mean(xc * xc, axis=-1, keepdims=True)
    y = xc * jax.lax.rsqrt(var + 1e-5)
    out_ref[...] = y * gamma_ref[...] + beta_ref[...]


def kernel(node_scalars, edge_feats, d, edge_index, W1, b1, W2, b2, gamma, beta):
    n_nodes, dim = node_scalars.shape
    e, f = edge_feats.shape
    r = d.shape[1]

    unit = _WINDOW * _WORKERS
    npad = ((2 * e + unit - 1) // unit) * unit
    idx = edge_index.reshape(2 * e)
    idx = jnp.pad(idx, (0, npad - 2 * e)).reshape(1, npad)

    g = _sc_gather(node_scalars, idx, npad, dim)

    nblocks = e // _BE
    out = pl.pallas_call(
        _mlp_body,
        grid=(nblocks,),
        in_specs=[
            pl.BlockSpec((_BE, dim), lambda i: (i, 0)),
            pl.BlockSpec((_BE, dim), lambda i: (i + nblocks, 0)),
            pl.BlockSpec((_BE, f), lambda i: (i, 0)),
            pl.BlockSpec((_BE, r), lambda i: (i, 0)),
            pl.BlockSpec(W1.shape, lambda i: (0, 0)),
            pl.BlockSpec((1, f), lambda i: (0, 0)),
            pl.BlockSpec(W2.shape, lambda i: (0, 0)),
            pl.BlockSpec((1, f), lambda i: (0, 0)),
            pl.BlockSpec((1, f), lambda i: (0, 0)),
            pl.BlockSpec((1, f), lambda i: (0, 0)),
        ],
        out_specs=pl.BlockSpec((_BE, f), lambda i: (i, 0)),
        out_shape=jax.ShapeDtypeStruct((e, f), jnp.float32),
        compiler_params=pltpu.CompilerParams(
            dimension_semantics=("arbitrary",),
        ),
    )(
        g, g, edge_feats, d, W1,
        b1.reshape(1, f), W2, b2.reshape(1, f),
        gamma.reshape(1, f), beta.reshape(1, f),
    )
    return out


# speedup vs baseline: 2.4041x; 1.2217x over previous
import functools

import jax
import jax.numpy as jnp
from jax.experimental import pallas as pl
from jax.experimental.pallas import tpu as pltpu
from jax.experimental.pallas import tpu_sc as plsc

_WINDOW = 128
_WORKERS = 32
_BE = 3200


def _sc_gather(table, idx2d, npad, dim):
    mesh = plsc.VectorSubcoreMesh(core_axis_name="core", subcore_axis_name="subcore")

    @functools.partial(
        pl.kernel,
        out_type=jax.ShapeDtypeStruct((npad, dim), table.dtype),
        mesh=mesh,
        compiler_params=pltpu.CompilerParams(use_tc_tiling_on_sc=False),
    )
    def gather_kernel(x_hbm, i_hbm, o_hbm):
        def body(i_vmem, o_vmem):
            pltpu.sync_copy(x_hbm.at[i_vmem.at[0]], o_vmem)

        pltpu.emit_pipeline(
            body,
            grid=(npad // _WINDOW,),
            in_specs=[pl.BlockSpec((1, _WINDOW), index_map=lambda i: (0, i))],
            out_specs=[pl.BlockSpec((_WINDOW, dim), index_map=lambda i: (i, 0))],
            core_axis_name=("core", "subcore"),
            dimension_semantics=(pltpu.PARALLEL,),
        )(i_hbm, o_hbm)

    return gather_kernel(table, idx2d)


def _mlp_body(g_ref, eft_ref, dt_ref, w1ab_ref, w1c_ref, w1d_ref, w2_ref,
              b1_ref, b2_ref, gamma_ref, beta_ref, out_ref):
    eft = eft_ref[...]
    x1 = (
        jax.lax.dot_general(w1ab_ref[...], g_ref[...],
                            (((1,), (1,)), ((), ())),
                            preferred_element_type=jnp.float32)
        + jnp.dot(w1c_ref[...], eft, preferred_element_type=jnp.float32)
        + jnp.dot(w1d_ref[...], dt_ref[...], preferred_element_type=jnp.float32)
        + b1_ref[...]
    )
    h = x1 * jax.nn.sigmoid(x1)
    x2 = jnp.dot(w2_ref[...], h, preferred_element_type=jnp.float32) + b2_ref[...]
    x = eft + x2 * jax.nn.sigmoid(x2)
    mu = jnp.mean(x, axis=0, keepdims=True)
    xc = x - mu
    var = jnp.mean(xc * xc, axis=0, keepdims=True)
    y = xc * jax.lax.rsqrt(var + 1e-5)
    out_ref[...] = y * gamma_ref[...] + beta_ref[...]


def kernel(node_scalars, edge_feats, d, edge_index, W1, b1, W2, b2, gamma, beta):
    n_nodes, dim = node_scalars.shape
    e, f = edge_feats.shape
    r = d.shape[1]

    unit = _WINDOW * _WORKERS
    npad = ((2 * e + unit - 1) // unit) * unit
    idx = edge_index.T.reshape(2 * e)
    idx = jnp.pad(idx, (0, npad - 2 * e)).reshape(1, npad)

    g = _sc_gather(node_scalars, idx, npad, dim)
    g128 = g.reshape(npad // 2, 2 * dim)

    w1t = W1.T
    w1ab = w1t[:, 0 : 2 * dim]
    w1c = w1t[:, 2 * dim : 2 * dim + f]
    w1d = w1t[:, 2 * dim + f :]

    nblocks = e // _BE
    out_t = pl.pallas_call(
        _mlp_body,
        grid=(nblocks,),
        in_specs=[
            pl.BlockSpec((_BE, 2 * dim), lambda i: (i, 0)),
            pl.BlockSpec((f, _BE), lambda i: (0, i)),
            pl.BlockSpec((r, _BE), lambda i: (0, i)),
            pl.BlockSpec((f, 2 * dim), lambda i: (0, 0)),
            pl.BlockSpec((f, f), lambda i: (0, 0)),
            pl.BlockSpec((f, r), lambda i: (0, 0)),
            pl.BlockSpec((f, f), lambda i: (0, 0)),
            pl.BlockSpec((f, 1), lambda i: (0, 0)),
            pl.BlockSpec((f, 1), lambda i: (0, 0)),
            pl.BlockSpec((f, 1), lambda i: (0, 0)),
            pl.BlockSpec((f, 1), lambda i: (0, 0)),
        ],
        out_specs=pl.BlockSpec((f, _BE), lambda i: (0, i)),
        out_shape=jax.ShapeDtypeStruct((f, e), jnp.float32),
        compiler_params=pltpu.CompilerParams(
            dimension_semantics=("arbitrary",),
        ),
    )(
        g128, edge_feats.T, d.T,
        w1ab, w1c, w1d, W2.T,
        b1.reshape(f, 1), b2.reshape(f, 1),
        gamma.reshape(f, 1), beta.reshape(f, 1),
    )
    return out_t.T


# speedup vs baseline: 4.9079x; 2.0415x over previous
import dataclasses
import functools

import jax
import jax.numpy as jnp
from jax.experimental import pallas as pl
from jax.experimental.pallas import tpu as pltpu
from jax.experimental.pallas import tpu_sc as plsc

_WINDOW = 128
_WORKERS = 32
_BE = 3200


def _sc_compiler_params():
    cp = pltpu.CompilerParams(use_tc_tiling_on_sc=False)
    if "needs_layout_passes" in pltpu.CompilerParams.__dataclass_fields__:
        cp = dataclasses.replace(cp, needs_layout_passes=False)
    return cp


def _sc_gather(table, eidx, npad, dim):
    mesh = plsc.VectorSubcoreMesh(core_axis_name="core", subcore_axis_name="subcore")
    half = _WINDOW // 2

    @functools.partial(
        pl.kernel,
        out_type=jax.ShapeDtypeStruct((npad, dim), table.dtype),
        mesh=mesh,
        scratch_types=[pltpu.VMEM((_WINDOW,), jnp.int32)],
        compiler_params=_sc_compiler_params(),
    )
    def gather_kernel(x_hbm, i_hbm, o_hbm, v_ref):
        def body(i_vmem, o_vmem):
            for c in range(half // 16):
                pos = jax.lax.iota(jnp.int32, 16) * 2 + c * 32
                plsc.store_scatter(v_ref, [pos], i_vmem.at[0][pl.ds(c * 16, 16)])
                plsc.store_scatter(v_ref, [pos + 1], i_vmem.at[1][pl.ds(c * 16, 16)])
            pltpu.sync_copy(x_hbm.at[v_ref], o_vmem)

        pltpu.emit_pipeline(
            body,
            grid=(npad // _WINDOW,),
            in_specs=[pl.BlockSpec((2, half), index_map=lambda i: (0, i))],
            out_specs=[pl.BlockSpec((_WINDOW, dim), index_map=lambda i: (i, 0))],
            core_axis_name=("core", "subcore"),
            dimension_semantics=(pltpu.PARALLEL,),
        )(i_hbm, o_hbm)

    return gather_kernel(table, eidx)


def _mlp_body(g_ref, eft_ref, dt_ref, w1ab_ref, w1c_ref, w1d_ref, w2_ref,
              b1_ref, b2_ref, gamma_ref, beta_ref, out_ref):
    eft = eft_ref[...]
    x1 = (
        jax.lax.dot_general(w1ab_ref[...], g_ref[...],
                            (((1,), (1,)), ((), ())),
                            preferred_element_type=jnp.float32)
        + jnp.dot(w1c_ref[...], eft, preferred_element_type=jnp.float32)
        + jnp.dot(w1d_ref[...], dt_ref[...], preferred_element_type=jnp.float32)
        + b1_ref[...]
    )
    h = x1 * jax.nn.sigmoid(x1)
    x2 = jnp.dot(w2_ref[...], h, preferred_element_type=jnp.float32) + b2_ref[...]
    x = eft + x2 * jax.nn.sigmoid(x2)
    mu = jnp.mean(x, axis=0, keepdims=True)
    xc = x - mu
    var = jnp.mean(xc * xc, axis=0, keepdims=True)
    y = xc * jax.lax.rsqrt(var + 1e-5)
    out_ref[...] = y * gamma_ref[...] + beta_ref[...]


def kernel(node_scalars, edge_feats, d, edge_index, W1, b1, W2, b2, gamma, beta):
    n_nodes, dim = node_scalars.shape
    e, f = edge_feats.shape
    r = d.shape[1]

    unit = _WINDOW * _WORKERS
    npad = ((2 * e + unit - 1) // unit) * unit
    eidx = jnp.pad(edge_index, ((0, 0), (0, npad // 2 - e)))

    g = _sc_gather(node_scalars, eidx, npad, dim)
    g128 = g.reshape(npad // 2, 2 * dim)

    w1t = W1.T
    w1ab = w1t[:, 0 : 2 * dim]
    w1c = w1t[:, 2 * dim : 2 * dim + f]
    w1d = w1t[:, 2 * dim + f :]

    nblocks = e // _BE
    out_t = pl.pallas_call(
        _mlp_body,
        grid=(nblocks,),
        in_specs=[
            pl.BlockSpec((_BE, 2 * dim), lambda i: (i, 0)),
            pl.BlockSpec((f, _BE), lambda i: (0, i)),
            pl.BlockSpec((r, _BE), lambda i: (0, i)),
            pl.BlockSpec((f, 2 * dim), lambda i: (0, 0)),
            pl.BlockSpec((f, f), lambda i: (0, 0)),
            pl.BlockSpec((f, r), lambda i: (0, 0)),
            pl.BlockSpec((f, f), lambda i: (0, 0)),
            pl.BlockSpec((f, 1), lambda i: (0, 0)),
            pl.BlockSpec((f, 1), lambda i: (0, 0)),
            pl.BlockSpec((f, 1), lambda i: (0, 0)),
            pl.BlockSpec((f, 1), lambda i: (0, 0)),
        ],
        out_specs=pl.BlockSpec((f, _BE), lambda i: (0, i)),
        out_shape=jax.ShapeDtypeStruct((f, e), jnp.float32),
        compiler_params=pltpu.CompilerParams(
            dimension_semantics=("arbitrary",),
        ),
    )(
        g128, edge_feats.T, d.T,
        w1ab, w1c, w1d, W2.T,
        b1.reshape(f, 1), b2.reshape(f, 1),
        gamma.reshape(f, 1), beta.reshape(f, 1),
    )
    return out_t.T
